# Initial kernel scaffold; baseline (speedup 1.0000x reference)
import jax
import jax.numpy as jnp
from jax.experimental import pallas as pl


def kernel(boxes, scores):
    raise NotImplementedError("write your pallas kernel here")


# speedup vs baseline: 498.9648x; 498.9648x over previous
import functools

import jax
import jax.numpy as jnp
from jax import lax
from jax.experimental import pallas as pl
from jax.experimental.pallas import tpu as pltpu
from jax.experimental.pallas import tpu_sc as plsc

N = 5000
NMS_T = 0.7
SCORE_T = 0.05
DETS = 100

L = 16
NSUB = 16
NPAD = 5120
PER = NPAD // NSUB
CH = PER // L
OUT_ROWS = 112
XOFF = 64

_f32 = jnp.float32


def _nms_body(x0h, y0h, x1h, y1h, sh, outh,
              x0v, y0v, x1v, y1v, av, kv, rowv, blkv, outv, shared):
    c_id = lax.axis_index("c")
    w = lax.axis_index("s")
    base = w * PER
    basef = (w * PER).astype(_f32)

    pltpu.sync_copy(x0h.at[pl.ds(base, PER)], x0v)
    pltpu.sync_copy(y0h.at[pl.ds(base, PER)], y0v)
    pltpu.sync_copy(x1h.at[pl.ds(base, PER)], x1v)
    pltpu.sync_copy(y1h.at[pl.ds(base, PER)], y1v)
    pltpu.sync_copy(sh.at[pl.ds(base, PER)], kv)

    li = lax.iota(jnp.int32, L)
    lif = li.astype(_f32)
    zeros = jnp.zeros((L,), _f32)

    for c in range(CH):
        sl = pl.ds(c * L, L)
        xa, ya, xb, yb = x0v[sl], y0v[sl], x1v[sl], y1v[sl]
        av[sl] = (xb - xa) * (yb - ya)
        s_c = kv[sl]
        gidx = basef + _f32(c * L) + lif
        invalid_key = _f32(-4.0) - gidx * _f32(1.0 / 5000.0)
        key = jnp.where(s_c > SCORE_T, s_c, invalid_key)
        key = jnp.where(s_c < _f32(-1e29), jnp.full((L,), _f32(-1e30)), key)
        kv[sl] = key

    def step(t, carry):
        bestv = jnp.full((L,), _f32(-3e30))
        besti = jnp.full((L,), _f32(1e9))
        for c in range(CH):
            sl = pl.ds(c * L, L)
            kc = kv[sl]
            idxc = basef + _f32(c * L) + lif
            cond = kc > bestv
            bestv = jnp.where(cond, kc, bestv)
            besti = jnp.where(cond, idxc, besti)
        mloc = jnp.max(bestv)
        iloc = jnp.min(jnp.where(bestv == mloc, besti, jnp.full((L,), _f32(1e9))))
        p = (iloc - basef).astype(jnp.int32)
        pv = jnp.full((L,), p, jnp.int32)
        wx0 = plsc.load_gather(x0v, [pv])
        wy0 = plsc.load_gather(y0v, [pv])
        wx1 = plsc.load_gather(x1v, [pv])
        wy1 = plsc.load_gather(y1v, [pv])

        mv = jnp.full((L,), mloc)
        iv = jnp.full((L,), iloc)
        row = jnp.where(li == 0, mv,
              jnp.where(li == 1, iv,
              jnp.where(li == 2, wx0,
              jnp.where(li == 3, wy0,
              jnp.where(li == 4, wx1, wy1)))))
        rowv[...] = row
        plsc.subcore_barrier()
        pltpu.sync_copy(rowv, shared.at[XOFF + w])
        plsc.subcore_barrier()
        pltpu.sync_copy(shared.at[pl.ds(XOFF, NSUB)], blkv)

        def col(j):
            return plsc.load_gather(blkv, [li, jnp.full((L,), j, jnp.int32)])

        maxes, gidxs = col(0), col(1)
        cx0, cy0, cx1, cy1 = col(2), col(3), col(4), col(5)
        gm = jnp.max(maxes)
        candm = maxes == gm
        gi = jnp.min(jnp.where(candm, gidxs, jnp.full((L,), _f32(1e9))))
        win = candm & (gidxs == gi)
        gx0 = jnp.sum(jnp.where(win, cx0, zeros))
        gy0 = jnp.sum(jnp.where(win, cy0, zeros))
        gx1 = jnp.sum(jnp.where(win, cx1, zeros))
        gy1 = jnp.sum(jnp.where(win, cy1, zeros))

        @pl.when(jnp.logical_and(w == 0, c_id == 0))
        def _emit():
            so = jnp.where(gm > SCORE_T, gm, _f32(0.0))
            orow = jnp.where(li == 0, jnp.full((L,), gx0),
                   jnp.where(li == 1, jnp.full((L,), gy0),
                   jnp.where(li == 2, jnp.full((L,), gx1),
                   jnp.where(li == 3, jnp.full((L,), gy1),
                   jnp.where(li == 4, jnp.full((L,), so), zeros)))))
            outv[pl.ds(t * L, L)] = orow

        wx0v = jnp.full((L,), gx0)
        wy0v = jnp.full((L,), gy0)
        wx1v = jnp.full((L,), gx1)
        wy1v = jnp.full((L,), gy1)
        wav = jnp.full((L,), (gx1 - gx0) * (gy1 - gy0))
        giv = jnp.full((L,), gi)
        for c in range(CH):
            sl = pl.ds(c * L, L)
            xa, ya, xb, yb = x0v[sl], y0v[sl], x1v[sl], y1v[sl]
            ac, kc = av[sl], kv[sl]
            iw = jnp.maximum(jnp.minimum(xb, wx1v) - jnp.maximum(xa, wx0v), zeros)
            ih = jnp.maximum(jnp.minimum(yb, wy1v) - jnp.maximum(ya, wy0v), zeros)
            inter = iw * ih
            union = jnp.maximum(ac + wav - inter, _f32(1e-9))
            sup = ((inter / union) > NMS_T) & (kc > SCORE_T)
            kc = jnp.where(sup, kc - _f32(2.0), kc)
            idxc = basef + _f32(c * L) + lif
            kv[sl] = jnp.where(idxc == giv, jnp.full((L,), _f32(-1e30)), kc)
        return carry

    lax.fori_loop(0, DETS, step, 0)

    @pl.when(jnp.logical_and(w == 0, c_id == 0))
    def _writeback():
        pltpu.sync_copy(outv, outh)


_nms = functools.partial(
    pl.kernel,
    out_type=jax.ShapeDtypeStruct((OUT_ROWS * L,), _f32),
    mesh=plsc.VectorSubcoreMesh(
        core_axis_name="c", subcore_axis_name="s", num_cores=2, num_subcores=NSUB
    ),
    scratch_types=[
        pltpu.VMEM((PER,), _f32),
        pltpu.VMEM((PER,), _f32),
        pltpu.VMEM((PER,), _f32),
        pltpu.VMEM((PER,), _f32),
        pltpu.VMEM((PER,), _f32),
        pltpu.VMEM((PER,), _f32),
        pltpu.VMEM((L,), _f32),
        pltpu.VMEM((NSUB, L), _f32),
        pltpu.VMEM((OUT_ROWS * L,), _f32),
        pltpu.VMEM_SHARED((XOFF + NSUB, L), _f32),
    ],
    compiler_params=pltpu.CompilerParams(needs_layout_passes=False),
)(_nms_body)


def kernel(boxes, scores):
    pad = NPAD - N
    x0 = jnp.pad(boxes[:, 0], (0, pad))
    y0 = jnp.pad(boxes[:, 1], (0, pad))
    x1 = jnp.pad(boxes[:, 2], (0, pad))
    y1 = jnp.pad(boxes[:, 3], (0, pad))
    s = jnp.pad(scores, (0, pad), constant_values=-1e30)
    out_flat = _nms(x0, y0, x1, y1, s)
    return out_flat.reshape(OUT_ROWS, L)[:DETS, :5]


# speedup vs baseline: 534.3485x; 1.0709x over previous
import functools

import jax
import jax.numpy as jnp
from jax import lax
from jax.experimental import pallas as pl
from jax.experimental.pallas import tpu as pltpu
from jax.experimental.pallas import tpu_sc as plsc

N = 5000
NMS_T = 0.7
SCORE_T = 0.05
DETS = 100

L = 16
NSUB = 16
NPAD = 5120
PER = NPAD // NSUB
CH = PER // L
OUT_ROWS = 112
XOFF = 64

_f32 = jnp.float32


def _nms_body(x0h, y0h, x1h, y1h, sh, outh,
              x0v, y0v, x1v, y1v, av, kv, iv_, rowv, blkv, outv, shared):
    c_id = lax.axis_index("c")
    w = lax.axis_index("s")
    base = w * PER
    basef = (w * PER).astype(_f32)

    pltpu.sync_copy(x0h.at[pl.ds(base, PER)], x0v)
    pltpu.sync_copy(y0h.at[pl.ds(base, PER)], y0v)
    pltpu.sync_copy(x1h.at[pl.ds(base, PER)], x1v)
    pltpu.sync_copy(y1h.at[pl.ds(base, PER)], y1v)
    pltpu.sync_copy(sh.at[pl.ds(base, PER)], kv)

    li = lax.iota(jnp.int32, L)
    lif = li.astype(_f32)
    zeros = jnp.zeros((L,), _f32)
    neg = jnp.full((L,), _f32(-1e30))
    big = jnp.full((L,), _f32(1e9))

    bestv0 = jnp.full((L,), _f32(-3e30))
    besti0 = big
    for c in range(CH):
        sl = pl.ds(c * L, L)
        xa, ya, xb, yb = x0v[sl], y0v[sl], x1v[sl], y1v[sl]
        av[sl] = (xb - xa) * (yb - ya)
        gidx = basef + _f32(c * L) + lif
        iv_[sl] = gidx
        s_c = kv[sl]
        invalid_key = _f32(-4.0) - gidx * _f32(1.0 / 5000.0)
        key = jnp.where(s_c > SCORE_T, s_c, invalid_key)
        key = jnp.where(s_c < _f32(-1e29), neg, key)
        kv[sl] = key
        cond = key > bestv0
        bestv0 = jnp.where(cond, key, bestv0)
        besti0 = jnp.where(cond, gidx, besti0)

    def step(t, carry):
        bestv, besti = carry
        mloc = jnp.max(bestv)
        iloc = jnp.min(jnp.where(bestv == mloc, besti, big))
        p = (iloc - basef).astype(jnp.int32)
        pv = jnp.full((L,), p, jnp.int32)
        wx0 = plsc.load_gather(x0v, [pv])
        wy0 = plsc.load_gather(y0v, [pv])
        wx1 = plsc.load_gather(x1v, [pv])
        wy1 = plsc.load_gather(y1v, [pv])

        mv = jnp.full((L,), mloc)
        ivv = jnp.full((L,), iloc)
        row = jnp.where(li == 0, mv,
              jnp.where(li == 1, ivv,
              jnp.where(li == 2, wx0,
              jnp.where(li == 3, wy0,
              jnp.where(li == 4, wx1, wy1)))))
        rowv[...] = row
        boff = XOFF + (t & 1) * NSUB
        pltpu.sync_copy(rowv, shared.at[boff + w])
        plsc.subcore_barrier()
        pltpu.sync_copy(shared.at[pl.ds(boff, NSUB)], blkv)

        def col(j):
            return plsc.load_gather(blkv, [li, jnp.full((L,), j, jnp.int32)])

        maxes, gidxs = col(0), col(1)
        cx0, cy0, cx1, cy1 = col(2), col(3), col(4), col(5)
        gm = jnp.max(maxes)
        candm = maxes == gm
        gi = jnp.min(jnp.where(candm, gidxs, big))
        win = candm & (gidxs == gi)
        gx0 = jnp.sum(jnp.where(win, cx0, zeros))
        gy0 = jnp.sum(jnp.where(win, cy0, zeros))
        gx1 = jnp.sum(jnp.where(win, cx1, zeros))
        gy1 = jnp.sum(jnp.where(win, cy1, zeros))

        @pl.when(jnp.logical_and(w == 0, c_id == 0))
        def _emit():
            so = jnp.where(gm > SCORE_T, gm, _f32(0.0))
            orow = jnp.where(li == 0, jnp.full((L,), gx0),
                   jnp.where(li == 1, jnp.full((L,), gy0),
                   jnp.where(li == 2, jnp.full((L,), gx1),
                   jnp.where(li == 3, jnp.full((L,), gy1),
                   jnp.where(li == 4, jnp.full((L,), so), zeros)))))
            outv[pl.ds(t * L, L)] = orow

        wx0v = jnp.full((L,), gx0)
        wy0v = jnp.full((L,), gy0)
        wx1v = jnp.full((L,), gx1)
        wy1v = jnp.full((L,), gy1)
        wav = jnp.full((L,), (gx1 - gx0) * (gy1 - gy0))
        giv = jnp.full((L,), gi)
        nbestv = jnp.full((L,), _f32(-3e30))
        nbesti = big
        for c in range(CH):
            sl = pl.ds(c * L, L)
            xa, ya, xb, yb = x0v[sl], y0v[sl], x1v[sl], y1v[sl]
            ac, kc, idxc = av[sl], kv[sl], iv_[sl]
            iw = jnp.maximum(jnp.minimum(xb, wx1v) - jnp.maximum(xa, wx0v), zeros)
            ih = jnp.maximum(jnp.minimum(yb, wy1v) - jnp.maximum(ya, wy0v), zeros)
            inter = iw * ih
            union = jnp.maximum(ac + wav - inter, _f32(1e-9))
            sup = (inter > NMS_T * union) & (kc > SCORE_T)
            kc = jnp.where(sup, kc - _f32(2.0), kc)
            kc = jnp.where(idxc == giv, neg, kc)
            kv[sl] = kc
            cond = kc > nbestv
            nbestv = jnp.where(cond, kc, nbestv)
            nbesti = jnp.where(cond, idxc, nbesti)
        return nbestv, nbesti

    lax.fori_loop(0, DETS, step, (bestv0, besti0))

    @pl.when(jnp.logical_and(w == 0, c_id == 0))
    def _writeback():
        pltpu.sync_copy(outv, outh)


_nms = functools.partial(
    pl.kernel,
    out_type=jax.ShapeDtypeStruct((OUT_ROWS * L,), _f32),
    mesh=plsc.VectorSubcoreMesh(
        core_axis_name="c", subcore_axis_name="s", num_cores=2, num_subcores=NSUB
    ),
    scratch_types=[
        pltpu.VMEM((PER,), _f32),
        pltpu.VMEM((PER,), _f32),
        pltpu.VMEM((PER,), _f32),
        pltpu.VMEM((PER,), _f32),
        pltpu.VMEM((PER,), _f32),
        pltpu.VMEM((PER,), _f32),
        pltpu.VMEM((PER,), _f32),
        pltpu.VMEM((L,), _f32),
        pltpu.VMEM((NSUB, L), _f32),
        pltpu.VMEM((OUT_ROWS * L,), _f32),
        pltpu.VMEM_SHARED((XOFF + 2 * NSUB, L), _f32),
    ],
    compiler_params=pltpu.CompilerParams(needs_layout_passes=False),
)(_nms_body)


def kernel(boxes, scores):
    pad = NPAD - N
    x0 = jnp.pad(boxes[:, 0], (0, pad))
    y0 = jnp.pad(boxes[:, 1], (0, pad))
    x1 = jnp.pad(boxes[:, 2], (0, pad))
    y1 = jnp.pad(boxes[:, 3], (0, pad))
    s = jnp.pad(scores, (0, pad), constant_values=-1e30)
    out_flat = _nms(x0, y0, x1, y1, s)
    return out_flat.reshape(OUT_ROWS, L)[:DETS, :5]


# speedup vs baseline: 542.0174x; 1.0144x over previous
import functools

import jax
import jax.numpy as jnp
from jax import lax
from jax.experimental import pallas as pl
from jax.experimental.pallas import tpu as pltpu
from jax.experimental.pallas import tpu_sc as plsc

N = 5000
NMS_T = 0.7
SCORE_T = 0.05
DETS = 100

L = 16
NSUB = 16
NPAD = 5120
PER = NPAD // NSUB
CH = PER // L
OUT_ROWS = 112
XOFF = 64

_f32 = jnp.float32


def _nms_body(x0h, y0h, x1h, y1h, sh, outh,
              x0f, y0f, x1f, y1f, sf,
              x0v, y0v, x1v, y1v, av, kv, iv_,
              rowv, blkv, tmpa, tmpb, outv, shared):
    c_id = lax.axis_index("c")
    w = lax.axis_index("s")
    base = w * PER
    basef = (w * PER).astype(_f32)

    pltpu.sync_copy(x0h, x0f)
    pltpu.sync_copy(y0h, y0f)
    pltpu.sync_copy(x1h, x1f)
    pltpu.sync_copy(y1h, y1f)
    pltpu.sync_copy(sh, sf)

    li = lax.iota(jnp.int32, L)
    zeros = jnp.zeros((L,), _f32)
    neg = jnp.full((L,), _f32(-1e30))

    bestv0 = jnp.full((L,), _f32(-3e30))
    besti0 = jnp.full((L,), _f32(1e9))
    for c in range(CH):
        sl = pl.ds(c * L, L)
        gidx32 = base + li * CH + c
        gidx = gidx32.astype(_f32)
        xa = plsc.load_gather(x0f, [gidx32])
        ya = plsc.load_gather(y0f, [gidx32])
        xb = plsc.load_gather(x1f, [gidx32])
        yb = plsc.load_gather(y1f, [gidx32])
        s_c = plsc.load_gather(sf, [gidx32])
        x0v[sl] = xa
        y0v[sl] = ya
        x1v[sl] = xb
        y1v[sl] = yb
        av[sl] = (xb - xa) * (yb - ya)
        iv_[sl] = gidx
        invalid_key = _f32(-4.0) - gidx * _f32(1.0 / 5000.0)
        key = jnp.where(s_c > SCORE_T, s_c, invalid_key)
        key = jnp.where(s_c < _f32(-1e29), neg, key)
        kv[sl] = key
        cond = key > bestv0
        bestv0 = jnp.where(cond, key, bestv0)
        besti0 = jnp.where(cond, gidx, besti0)

    def step(t, carry):
        bestv, besti = carry
        mloc = jnp.max(bestv)
        plsc.store_compressed(tmpa.at[...], besti, mask=bestv == mloc)
        iloc = tmpa[...][0]

        rowv[...] = jnp.where(li == 0, jnp.full((L,), mloc), jnp.full((L,), iloc))
        boff = XOFF + (t & 1) * NSUB
        pltpu.sync_copy(rowv, shared.at[boff + w])
        plsc.subcore_barrier()
        pltpu.sync_copy(shared.at[pl.ds(boff, NSUB)], blkv)

        maxes = plsc.load_gather(blkv, [li, jnp.zeros((L,), jnp.int32)])
        gidxs = plsc.load_gather(blkv, [li, jnp.full((L,), 1, jnp.int32)])
        gm = jnp.max(maxes)
        plsc.store_compressed(tmpb.at[...], gidxs, mask=maxes == gm)
        gi = tmpb[...][0]

        pv = jnp.full((L,), gi.astype(jnp.int32))
        wx0v = plsc.load_gather(x0f, [pv])
        wy0v = plsc.load_gather(y0f, [pv])
        wx1v = plsc.load_gather(x1f, [pv])
        wy1v = plsc.load_gather(y1f, [pv])
        wav = (wx1v - wx0v) * (wy1v - wy0v)
        giv = jnp.full((L,), gi)

        @pl.when(jnp.logical_and(w == 0, c_id == 0))
        def _emit():
            so = jnp.where(gm > SCORE_T, gm, _f32(0.0))
            orow = jnp.where(li == 0, wx0v,
                   jnp.where(li == 1, wy0v,
                   jnp.where(li == 2, wx1v,
                   jnp.where(li == 3, wy1v,
                   jnp.where(li == 4, jnp.full((L,), so), zeros)))))
            outv[pl.ds(t * L, L)] = orow

        nbestv = jnp.full((L,), _f32(-3e30))
        nbesti = jnp.full((L,), _f32(1e9))
        for c in range(CH):
            sl = pl.ds(c * L, L)
            xa, ya, xb, yb = x0v[sl], y0v[sl], x1v[sl], y1v[sl]
            ac, kc, idxc = av[sl], kv[sl], iv_[sl]
            iw = jnp.maximum(jnp.minimum(xb, wx1v) - jnp.maximum(xa, wx0v), zeros)
            ih = jnp.maximum(jnp.minimum(yb, wy1v) - jnp.maximum(ya, wy0v), zeros)
            inter = iw * ih
            union = jnp.maximum(ac + wav - inter, _f32(1e-9))
            sup = (inter > NMS_T * union) & (kc > SCORE_T)
            kc = jnp.where(sup, kc - _f32(2.0), kc)
            kc = jnp.where(idxc == giv, neg, kc)
            kv[sl] = kc
            cond = kc > nbestv
            nbestv = jnp.where(cond, kc, nbestv)
            nbesti = jnp.where(cond, idxc, nbesti)
        return nbestv, nbesti

    lax.fori_loop(0, DETS, step, (bestv0, besti0))

    @pl.when(jnp.logical_and(w == 0, c_id == 0))
    def _writeback():
        pltpu.sync_copy(outv, outh)


_nms = functools.partial(
    pl.kernel,
    out_type=jax.ShapeDtypeStruct((OUT_ROWS * L,), _f32),
    mesh=plsc.VectorSubcoreMesh(
        core_axis_name="c", subcore_axis_name="s", num_cores=2, num_subcores=NSUB
    ),
    scratch_types=[
        pltpu.VMEM((NPAD,), _f32),
        pltpu.VMEM((NPAD,), _f32),
        pltpu.VMEM((NPAD,), _f32),
        pltpu.VMEM((NPAD,), _f32),
        pltpu.VMEM((NPAD,), _f32),
        pltpu.VMEM((PER,), _f32),
        pltpu.VMEM((PER,), _f32),
        pltpu.VMEM((PER,), _f32),
        pltpu.VMEM((PER,), _f32),
        pltpu.VMEM((PER,), _f32),
        pltpu.VMEM((PER,), _f32),
        pltpu.VMEM((PER,), _f32),
        pltpu.VMEM((L,), _f32),
        pltpu.VMEM((NSUB, L), _f32),
        pltpu.VMEM((L,), _f32),
        pltpu.VMEM((L,), _f32),
        pltpu.VMEM((OUT_ROWS * L,), _f32),
        pltpu.VMEM_SHARED((XOFF + 2 * NSUB, L), _f32),
    ],
    compiler_params=pltpu.CompilerParams(needs_layout_passes=False),
)(_nms_body)


def kernel(boxes, scores):
    pad = NPAD - N
    x0 = jnp.pad(boxes[:, 0], (0, pad))
    y0 = jnp.pad(boxes[:, 1], (0, pad))
    x1 = jnp.pad(boxes[:, 2], (0, pad))
    y1 = jnp.pad(boxes[:, 3], (0, pad))
    s = jnp.pad(scores, (0, pad), constant_values=-1e30)
    out_flat = _nms(x0, y0, x1, y1, s)
    return out_flat.reshape(OUT_ROWS, L)[:DETS, :5]


# speedup vs baseline: 657.6475x; 1.2133x over previous
import functools

import jax
import jax.numpy as jnp
from jax import lax
from jax.experimental import pallas as pl
from jax.experimental.pallas import tpu as pltpu
from jax.experimental.pallas import tpu_sc as plsc

N = 5000
NMS_T = 0.7
SCORE_T = 0.05
DETS = 100

L = 16
NSUB = 16
NPAD = 5120
PER = NPAD // NSUB
CH = PER // L
OUT_ROWS = 112
XOFF = 64

_f32 = jnp.float32


def _nms_body(x0h, y0h, x1h, y1h, sh, outh,
              x0f, y0f, x1f, y1f, sf,
              x0v, y0v, x1v, y1v, av, kv, iv_,
              rowv, blkv, tmpa, tmpb, outv, shared):
    c_id = lax.axis_index("c")
    w = lax.axis_index("s")
    base = w * PER
    basef = (w * PER).astype(_f32)

    pltpu.sync_copy(x0h, x0f)
    pltpu.sync_copy(y0h, y0f)
    pltpu.sync_copy(x1h, x1f)
    pltpu.sync_copy(y1h, y1f)
    pltpu.sync_copy(sh, sf)

    li = lax.iota(jnp.int32, L)
    zeros = jnp.zeros((L,), _f32)
    neg = jnp.full((L,), _f32(-1e30))

    b1v0 = jnp.full((L,), _f32(-3e30))
    b1i0 = jnp.full((L,), _f32(1e9))
    b2v0 = jnp.full((L,), _f32(-3e30))
    b2i0 = jnp.full((L,), _f32(1e9))
    for c in range(CH):
        sl = pl.ds(c * L, L)
        gidx32 = base + li * CH + c
        gidx = gidx32.astype(_f32)
        xa = plsc.load_gather(x0f, [gidx32])
        ya = plsc.load_gather(y0f, [gidx32])
        xb = plsc.load_gather(x1f, [gidx32])
        yb = plsc.load_gather(y1f, [gidx32])
        s_c = plsc.load_gather(sf, [gidx32])
        x0v[sl] = xa
        y0v[sl] = ya
        x1v[sl] = xb
        y1v[sl] = yb
        av[sl] = (xb - xa) * (yb - ya)
        iv_[sl] = gidx
        invalid_key = _f32(-4.0) - gidx * _f32(1.0 / 5000.0)
        key = jnp.where(s_c > SCORE_T, s_c, invalid_key)
        key = jnp.where(s_c < _f32(-1e29), neg, key)
        kv[sl] = key
        gt1 = key > b1v0
        gt2 = key > b2v0
        b2v0 = jnp.where(gt1, b1v0, jnp.where(gt2, key, b2v0))
        b2i0 = jnp.where(gt1, b1i0, jnp.where(gt2, gidx, b2i0))
        b1v0 = jnp.where(gt1, key, b1v0)
        b1i0 = jnp.where(gt1, gidx, b1i0)

    def cond_fn(carry):
        return carry[0] < DETS

    def step(carry):
        t, r, b1v, b1i, b2v, b2i = carry
        m1 = jnp.max(b1v)
        plsc.store_compressed(tmpa.at[...], b1i, mask=b1v == m1)
        i1 = tmpa[...][0]
        winlane = b1i == jnp.full((L,), i1)
        rb1v = jnp.where(winlane, b2v, b1v)
        rb1i = jnp.where(winlane, b2i, b1i)
        m2 = jnp.max(rb1v)
        plsc.store_compressed(tmpb.at[...], rb1i, mask=rb1v == m2)
        i2 = tmpb[...][0]

        rowv[...] = jnp.where(li == 0, jnp.full((L,), m1),
                    jnp.where(li == 1, jnp.full((L,), i1),
                    jnp.where(li == 2, jnp.full((L,), m2), jnp.full((L,), i2))))
        boff = XOFF + (r & 1) * NSUB
        pltpu.sync_copy(rowv, shared.at[boff + w])
        plsc.subcore_barrier()
        pltpu.sync_copy(shared.at[pl.ds(boff, NSUB)], blkv)

        def col(j):
            return plsc.load_gather(blkv, [li, jnp.full((L,), j, jnp.int32)])

        cm1, ci1, cm2, ci2 = col(0), col(1), col(2), col(3)
        gm1 = jnp.max(cm1)
        plsc.store_compressed(tmpa.at[...], ci1, mask=cm1 == gm1)
        gi1 = tmpa[...][0]
        p1 = jnp.full((L,), gi1.astype(jnp.int32))
        w1x0 = plsc.load_gather(x0f, [p1])
        w1y0 = plsc.load_gather(y0f, [p1])
        w1x1 = plsc.load_gather(x1f, [p1])
        w1y1 = plsc.load_gather(y1f, [p1])
        w1a = (w1x1 - w1x0) * (w1y1 - w1y0)

        def supp_by_w1(ix32, val):
            ex0 = plsc.load_gather(x0f, [ix32])
            ey0 = plsc.load_gather(y0f, [ix32])
            ex1 = plsc.load_gather(x1f, [ix32])
            ey1 = plsc.load_gather(y1f, [ix32])
            ea = (ex1 - ex0) * (ey1 - ey0)
            iw = jnp.maximum(jnp.minimum(ex1, w1x1) - jnp.maximum(ex0, w1x0), zeros)
            ih = jnp.maximum(jnp.minimum(ey1, w1y1) - jnp.maximum(ey0, w1y0), zeros)
            inter = iw * ih
            union = jnp.maximum(ea + w1a - inter, _f32(1e-9))
            return (inter > NMS_T * union) & (val > SCORE_T)

        gi1v = jnp.full((L,), gi1)
        avail1 = jnp.logical_not(supp_by_w1(ci1.astype(jnp.int32), cm1)) & (ci1 != gi1v)
        avail2 = jnp.logical_not(supp_by_w1(ci2.astype(jnp.int32), cm2)) & (ci2 != gi1v)
        rep_v = jnp.where(avail1, cm1, jnp.where(avail2, cm2, jnp.full((L,), _f32(-3e30))))
        rep_i = jnp.where(avail1, ci1, jnp.where(avail2, ci2, jnp.full((L,), _f32(1e9))))
        gm2 = jnp.max(rep_v)
        plsc.store_compressed(tmpb.at[...], rep_i, mask=rep_v == gm2)
        gi2 = tmpb[...][0]
        fb_lane = jnp.logical_not(avail1) & jnp.logical_not(avail2) & (cm2 >= gm2)
        fb = jnp.any(fb_lane)
        ok2 = jnp.logical_not(fb)

        p2r = jnp.full((L,), jnp.minimum(gi2, _f32(NPAD - 1)).astype(jnp.int32))
        w2x0r = plsc.load_gather(x0f, [p2r])
        w2y0r = plsc.load_gather(y0f, [p2r])
        w2x1r = plsc.load_gather(x1f, [p2r])
        w2y1r = plsc.load_gather(y1f, [p2r])
        w2x0 = jnp.where(ok2, w2x0r, neg)
        w2y0 = jnp.where(ok2, w2y0r, neg)
        w2x1 = jnp.where(ok2, w2x1r, neg)
        w2y1 = jnp.where(ok2, w2y1r, neg)
        w2a = (w2x1 - w2x0) * (w2y1 - w2y0)
        gi2v = jnp.where(ok2, jnp.full((L,), gi2), jnp.full((L,), _f32(-5.0)))

        @pl.when(jnp.logical_and(w == 0, c_id == 0))
        def _emit():
            so1 = jnp.where(gm1 > SCORE_T, gm1, _f32(0.0))
            orow = jnp.where(li == 0, w1x0,
                   jnp.where(li == 1, w1y0,
                   jnp.where(li == 2, w1x1,
                   jnp.where(li == 3, w1y1,
                   jnp.where(li == 4, jnp.full((L,), so1), zeros)))))
            outv[pl.ds(t * L, L)] = orow

        @pl.when(jnp.logical_and(jnp.logical_and(w == 0, c_id == 0),
                                 jnp.logical_and(ok2, t + 1 < DETS)))
        def _emit2():
            so2 = jnp.where(gm2 > SCORE_T, gm2, _f32(0.0))
            orow2 = jnp.where(li == 0, w2x0,
                    jnp.where(li == 1, w2y0,
                    jnp.where(li == 2, w2x1,
                    jnp.where(li == 3, w2y1,
                    jnp.where(li == 4, jnp.full((L,), so2), zeros)))))
            outv[pl.ds((t + 1) * L, L)] = orow2

        nb1v = jnp.full((L,), _f32(-3e30))
        nb1i = jnp.full((L,), _f32(1e9))
        nb2v = jnp.full((L,), _f32(-3e30))
        nb2i = jnp.full((L,), _f32(1e9))
        for c in range(CH):
            sl = pl.ds(c * L, L)
            xa, ya, xb, yb = x0v[sl], y0v[sl], x1v[sl], y1v[sl]
            ac, kc, idxc = av[sl], kv[sl], iv_[sl]
            iw1 = jnp.maximum(jnp.minimum(xb, w1x1) - jnp.maximum(xa, w1x0), zeros)
            ih1 = jnp.maximum(jnp.minimum(yb, w1y1) - jnp.maximum(ya, w1y0), zeros)
            in1 = iw1 * ih1
            un1 = jnp.maximum(ac + w1a - in1, _f32(1e-9))
            iw2 = jnp.maximum(jnp.minimum(xb, w2x1) - jnp.maximum(xa, w2x0), zeros)
            ih2 = jnp.maximum(jnp.minimum(yb, w2y1) - jnp.maximum(ya, w2y0), zeros)
            in2 = iw2 * ih2
            un2 = jnp.maximum(ac + w2a - in2, _f32(1e-9))
            sup = ((in1 > NMS_T * un1) | (in2 > NMS_T * un2)) & (kc > SCORE_T)
            kc = jnp.where(sup, kc - _f32(2.0), kc)
            kc = jnp.where((idxc == jnp.full((L,), gi1)) | (idxc == gi2v), neg, kc)
            kv[sl] = kc
            gt1 = kc > nb1v
            gt2 = kc > nb2v
            nb2v = jnp.where(gt1, nb1v, jnp.where(gt2, kc, nb2v))
            nb2i = jnp.where(gt1, nb1i, jnp.where(gt2, idxc, nb2i))
            nb1v = jnp.where(gt1, kc, nb1v)
            nb1i = jnp.where(gt1, idxc, nb1i)
        tn = t + jnp.where(ok2, jnp.int32(2), jnp.int32(1))
        return tn, r + jnp.int32(1), nb1v, nb1i, nb2v, nb2i

    lax.while_loop(cond_fn, step,
                   (jnp.int32(0), jnp.int32(0), b1v0, b1i0, b2v0, b2i0))

    @pl.when(jnp.logical_and(w == 0, c_id == 0))
    def _writeback():
        pltpu.sync_copy(outv, outh)


_nms = functools.partial(
    pl.kernel,
    out_type=jax.ShapeDtypeStruct((OUT_ROWS * L,), _f32),
    mesh=plsc.VectorSubcoreMesh(
        core_axis_name="c", subcore_axis_name="s", num_cores=2, num_subcores=NSUB
    ),
    scratch_types=[
        pltpu.VMEM((NPAD,), _f32),
        pltpu.VMEM((NPAD,), _f32),
        pltpu.VMEM((NPAD,), _f32),
        pltpu.VMEM((NPAD,), _f32),
        pltpu.VMEM((NPAD,), _f32),
        pltpu.VMEM((PER,), _f32),
        pltpu.VMEM((PER,), _f32),
        pltpu.VMEM((PER,), _f32),
        pltpu.VMEM((PER,), _f32),
        pltpu.VMEM((PER,), _f32),
        pltpu.VMEM((PER,), _f32),
        pltpu.VMEM((PER,), _f32),
        pltpu.VMEM((L,), _f32),
        pltpu.VMEM((NSUB, L), _f32),
        pltpu.VMEM((L,), _f32),
        pltpu.VMEM((L,), _f32),
        pltpu.VMEM((OUT_ROWS * L,), _f32),
        pltpu.VMEM_SHARED((XOFF + 2 * NSUB, L), _f32),
    ],
    compiler_params=pltpu.CompilerParams(needs_layout_passes=False),
)(_nms_body)


def kernel(boxes, scores):
    pad = NPAD - N
    x0 = jnp.pad(boxes[:, 0], (0, pad))
    y0 = jnp.pad(boxes[:, 1], (0, pad))
    x1 = jnp.pad(boxes[:, 2], (0, pad))
    y1 = jnp.pad(boxes[:, 3], (0, pad))
    s = jnp.pad(scores, (0, pad), constant_values=-1e30)
    out_flat = _nms(x0, y0, x1, y1, s)
    return out_flat.reshape(OUT_ROWS, L)[:DETS, :5]
